# Initial kernel scaffold; baseline (speedup 1.0000x reference)
import jax
import jax.numpy as jnp
from jax.experimental import pallas as pl


def kernel(relative_position_bias_table, relative_position_index):
    raise NotImplementedError("write your pallas kernel here")


# speedup vs baseline: 2.8486x; 2.8486x over previous
import functools

import jax
import jax.numpy as jnp
from jax import lax
from jax.experimental import pallas as pl
from jax.experimental.pallas import tpu as pltpu
from jax.experimental.pallas import tpu_sc as plsc


_H = 32
_NPOS = 256 * 256
_NW = 32
_PPW = _NPOS // _NW
_SUB = 1024
_NSUB = _PPW // _SUB
_VECS = _SUB // 16


def _body(tab_hbm, idx_hbm, out_hbm, tab_v, idx_v, out_v0, out_v1, sem0, sem1):
    wid = lax.axis_index("s") * 2 + lax.axis_index("c")
    base = wid * _PPW

    pltpu.sync_copy(tab_hbm, tab_v)
    pltpu.sync_copy(idx_hbm.at[pl.ds(base, _PPW)], idx_v)

    bufs = (out_v0, out_v1)
    sems = (sem0, sem1)
    copies = [None, None]
    for sub in range(_NSUB):
        buf = bufs[sub % 2]
        if copies[sub % 2] is not None:
            copies[sub % 2].wait()

        def fill(pv, carry, sub=sub, buf=buf):
            off = pv * 16
            iv = idx_v[pl.ds(sub * _SUB + off, 16)]
            iv = iv * _H
            for h in range(_H):
                buf[h, pl.ds(off, 16)] = plsc.load_gather(tab_v, [iv + h])
            return carry

        lax.fori_loop(0, _VECS, fill, 0)

        dst = out_hbm.at[:, pl.ds(base + sub * _SUB, _SUB)]
        copies[sub % 2] = pltpu.async_copy(buf, dst, sems[sub % 2])

    for c in copies:
        if c is not None:
            c.wait()


def _gather_all(tab_flat, idx_flat):
    mesh = plsc.VectorSubcoreMesh(core_axis_name="c", subcore_axis_name="s")
    run = functools.partial(
        pl.kernel,
        mesh=mesh,
        out_type=jax.ShapeDtypeStruct((_H, _NPOS), jnp.float32),
        scratch_types=[
            pltpu.VMEM((tab_flat.shape[0],), jnp.float32),
            pltpu.VMEM((_PPW,), jnp.int32),
            pltpu.VMEM((_H, _SUB), jnp.float32),
            pltpu.VMEM((_H, _SUB), jnp.float32),
            pltpu.SemaphoreType.DMA,
            pltpu.SemaphoreType.DMA,
        ],
        compiler_params=pltpu.CompilerParams(needs_layout_passes=False),
    )(_body)
    return run(tab_flat, idx_flat)


def kernel(relative_position_bias_table, relative_position_index):
    n = relative_position_index.shape[0]
    tab_flat = relative_position_bias_table.reshape(-1)
    idx_flat = relative_position_index.reshape(-1).astype(jnp.int32)
    out = _gather_all(tab_flat, idx_flat)
    return out.reshape(_H, n, n)


# speedup vs baseline: 6.9349x; 2.4345x over previous
import functools

import jax
import jax.numpy as jnp
from jax import lax
from jax.experimental import pallas as pl
from jax.experimental.pallas import tpu as pltpu
from jax.experimental.pallas import tpu_sc as plsc


_H = 32
_N = 256
_NW = 32
_RPW = _N // _NW
_RSUB = 2
_NSUB = _RPW // _RSUB
_TPAD = 1024


def _body(tab_hbm, idx_hbm, out_hbm, tab_v, idx_v, b0, b1, s0, s1):
    wid = lax.axis_index("s") * 2 + lax.axis_index("c")
    row0 = wid * _RPW

    pltpu.sync_copy(tab_hbm, tab_v)
    pltpu.sync_copy(idx_hbm.at[pl.ds(row0, _RPW)], idx_v)

    bufs = (b0, b1)
    sems = (s0, s1)
    copies = [None, None]
    for sub in range(_NSUB):
        buf = bufs[sub % 2]
        if copies[sub % 2] is not None:
            copies[sub % 2].wait()

        for r in range(_RSUB):
            row = sub * _RSUB + r

            @plsc.parallel_loop(0, _N // 16)
            def fill(pv, row=row, r=r, buf=buf):
                iv = idx_v[row, pl.ds(pv * 16, 16)]
                for h in range(_H):
                    buf[h, r, pl.ds(pv * 16, 16)] = plsc.load_gather(
                        tab_v, [iv + h * _TPAD]
                    )

        dst = out_hbm.at[:, pl.ds(row0 + sub * _RSUB, _RSUB), :]
        copies[sub % 2] = pltpu.async_copy(buf, dst, sems[sub % 2])

    for c in copies:
        if c is not None:
            c.wait()


def _gather_all(tab_t, idx):
    mesh = plsc.VectorSubcoreMesh(core_axis_name="c", subcore_axis_name="s")
    run = functools.partial(
        pl.kernel,
        mesh=mesh,
        out_type=jax.ShapeDtypeStruct((_H, _N, _N), jnp.float32),
        scratch_types=[
            pltpu.VMEM((_H * _TPAD,), jnp.float32),
            pltpu.VMEM((_RPW, _N), jnp.int32),
            pltpu.VMEM((_H, _RSUB, _N), jnp.float32),
            pltpu.VMEM((_H, _RSUB, _N), jnp.float32),
            pltpu.SemaphoreType.DMA,
            pltpu.SemaphoreType.DMA,
        ],
        compiler_params=pltpu.CompilerParams(needs_layout_passes=False),
    )(_body)
    return run(tab_t, idx)


def kernel(relative_position_bias_table, relative_position_index):
    nbins = relative_position_bias_table.shape[0]
    tab_t = jnp.zeros((_H, _TPAD), jnp.float32)
    tab_t = tab_t.at[:, :nbins].set(relative_position_bias_table.T)
    idx = relative_position_index.astype(jnp.int32)
    return _gather_all(tab_t.reshape(-1), idx)


# speedup vs baseline: 7.4597x; 1.0757x over previous
import functools

import jax
import jax.numpy as jnp
from jax import lax
from jax.experimental import pallas as pl
from jax.experimental.pallas import tpu as pltpu
from jax.experimental.pallas import tpu_sc as plsc


_H = 32
_N = 256
_NW = 32
_RPW = _N // _NW
_RSUB = 2
_NSUB = _RPW // _RSUB
_TPAD = 1024


def _body(tab_hbm, idx_hbm, out_hbm, tab_v, idx_v, b0, b1, s0, s1):
    wid = lax.axis_index("s") * 2 + lax.axis_index("c")
    row0 = wid * _RPW

    pltpu.sync_copy(tab_hbm, tab_v)
    pltpu.sync_copy(idx_hbm.at[pl.ds(row0, _RPW)], idx_v)

    bufs = (b0, b1)
    sems = (s0, s1)
    copies = [None, None]
    for sub in range(_NSUB):
        buf = bufs[sub % 2]
        if copies[sub % 2] is not None:
            copies[sub % 2].wait()

        @plsc.parallel_loop(0, _RSUB * (_N // 16), unroll=2)
        def fill(pv, sub=sub, buf=buf):
            r = pv // (_N // 16)
            off = (pv % (_N // 16)) * 16
            row = sub * _RSUB + r
            iv = idx_v[row, pl.ds(off, 16)]
            for h in range(_H):
                buf[h, r, pl.ds(off, 16)] = plsc.load_gather(
                    tab_v, [iv + h * _TPAD]
                )

        dst = out_hbm.at[:, pl.ds(row0 + sub * _RSUB, _RSUB), :]
        copies[sub % 2] = pltpu.async_copy(buf, dst, sems[sub % 2])

    for c in copies:
        if c is not None:
            c.wait()


def _gather_all(tab_t, idx):
    mesh = plsc.VectorSubcoreMesh(core_axis_name="c", subcore_axis_name="s")
    run = functools.partial(
        pl.kernel,
        mesh=mesh,
        out_type=jax.ShapeDtypeStruct((_H, _N, _N), jnp.float32),
        scratch_types=[
            pltpu.VMEM((_H * _TPAD,), jnp.float32),
            pltpu.VMEM((_RPW, _N), jnp.int32),
            pltpu.VMEM((_H, _RSUB, _N), jnp.float32),
            pltpu.VMEM((_H, _RSUB, _N), jnp.float32),
            pltpu.SemaphoreType.DMA,
            pltpu.SemaphoreType.DMA,
        ],
        compiler_params=pltpu.CompilerParams(needs_layout_passes=False),
    )(_body)
    return run(tab_t, idx)


def kernel(relative_position_bias_table, relative_position_index):
    nbins = relative_position_bias_table.shape[0]
    tab_t = jnp.zeros((_H, _TPAD), jnp.float32)
    tab_t = tab_t.at[:, :nbins].set(relative_position_bias_table.T)
    idx = relative_position_index.astype(jnp.int32)
    return _gather_all(tab_t.reshape(-1), idx)
